# Initial kernel scaffold; baseline (speedup 1.0000x reference)
import jax
import jax.numpy as jnp
from jax.experimental import pallas as pl


def kernel(op_x, feat_x, cond1_x, cond2_x, bitmap_x, has_cond, W_op, b_op, W_pred, b_pred, W_bm, b_bm, W_feat, b_feat, W_r1, b_r1, W_r2, b_r2, W_r3, b_r3, W_h21, b_h21, W_h31, b_h31, W_o1, b_o1, W_h22, b_h22, W_h32, b_h32, W_o2, b_o2):
    raise NotImplementedError("write your pallas kernel here")


# speedup vs baseline: 2.5727x; 2.5727x over previous
import functools

import jax
import jax.numpy as jnp
from jax.experimental import pallas as pl
from jax.experimental.pallas import tpu as pltpu

_B = 128
_D = 5
_N = 31
_OP = 16
_PRED = 512
_FEAT = 64
_HID = 128
_BITMAP = 1000
_REP = 128

_BT = 32


def _dot(a, b):
    return jax.lax.dot_general(
        a, b, (((1,), (0,)), ((), ())), preferred_element_type=jnp.float32
    )


def _tree_body(
    op_ref, feat_ref, c1_ref, c2_ref, bm_ref, hc_ref,
    W_op_ref, b_op_ref, W_pred_ref, b_pred_ref, W_bm_ref, b_bm_ref,
    W_feat_ref, b_feat_ref, W_r1_ref, b_r1_ref, W_r2_ref, b_r2_ref,
    W_r3_ref, b_r3_ref, W_h21_ref, b_h21_ref, W_h31_ref, b_h31_ref,
    W_o1_ref, b_o1_ref, W_h22_ref, b_h22_ref, W_h32_ref, b_h32_ref,
    W_o2_ref, b_o2_ref,
    cost_ref, card_ref,
):
    M = _BT * _N

    op_v = _dot(op_ref[...].reshape(M, _OP), W_op_ref[...]) + b_op_ref[...]
    feat_v = _dot(feat_ref[...].reshape(M, _FEAT), W_feat_ref[...]) + b_feat_ref[...]
    Wp = W_pred_ref[...]
    bp = b_pred_ref[...]
    c1 = _dot(c1_ref[...].reshape(M, _PRED), Wp) + bp
    c2 = _dot(c2_ref[...].reshape(M, _PRED), Wp) + bp
    bmv = _dot(bm_ref[...].reshape(M, _BITMAP), W_bm_ref[...]) + b_bm_ref[...]
    bm = bmv * hc_ref[...].reshape(M, _HID)

    Wr1 = W_r1_ref[...]
    z = _dot(op_v, Wr1[0 * _HID:1 * _HID])
    z = z + _dot(feat_v, Wr1[1 * _HID:2 * _HID])
    z = z + _dot(c1, Wr1[2 * _HID:3 * _HID])
    z = z + _dot(c2, Wr1[3 * _HID:4 * _HID])
    z = z + _dot(bm, Wr1[4 * _HID:5 * _HID])
    z = z + b_r1_ref[...]
    zb = z.reshape(_BT, _N, 512)

    cbase = 5 * _HID
    Wlr = Wr1[cbase + 0 * _REP: cbase + 1 * _REP]
    Wrr = Wr1[cbase + 1 * _REP: cbase + 2 * _REP]
    Wll = Wr1[cbase + 2 * _REP: cbase + 3 * _REP]
    Wlrt = Wr1[cbase + 3 * _REP: cbase + 4 * _REP]
    Wrl = Wr1[cbase + 4 * _REP: cbase + 5 * _REP]
    Wrrt = Wr1[cbase + 5 * _REP: cbase + 6 * _REP]

    W2 = W_r2_ref[...]
    b2 = b_r2_ref[...]
    W3 = W_r3_ref[...]
    b3 = b_r3_ref[...]

    reps = [None] * _D
    for l in range(_D - 1, -1, -1):
        n = 1 << l
        a = n - 1
        zl = zb[:, a:a + n, :].reshape(_BT * n, 512)
        if l <= _D - 2:
            C = reps[l + 1].reshape(_BT, n, 2, _REP)
            left = C[:, :, 0, :].reshape(_BT * n, _REP)
            right = C[:, :, 1, :].reshape(_BT * n, _REP)
            zl = zl + _dot(left, Wlr) + _dot(right, Wrr)
        if l <= _D - 3:
            G = reps[l + 2].reshape(_BT, n, 4, _REP)
            gc0 = G[:, :, 0, :].reshape(_BT * n, _REP)
            gc1 = G[:, :, 1, :].reshape(_BT * n, _REP)
            gc2 = G[:, :, 2, :].reshape(_BT * n, _REP)
            gc3 = G[:, :, 3, :].reshape(_BT * n, _REP)
            zl = (zl + _dot(gc0, Wll) + _dot(gc1, Wlrt)
                  + _dot(gc2, Wrl) + _dot(gc3, Wrrt))
        h = jnp.maximum(zl, 0.0)
        h = jnp.maximum(_dot(h, W2) + b2, 0.0)
        h = jnp.maximum(_dot(h, W3) + b3, 0.0)
        reps[l] = h.reshape(_BT, n, _REP)

    root = reps[0].reshape(_BT, _REP)
    cost = jnp.maximum(_dot(root, W_h21_ref[...]) + b_h21_ref[...], 0.0)
    cost = jnp.maximum(_dot(cost, W_h31_ref[...]) + b_h31_ref[...], 0.0)
    cost = jax.nn.sigmoid(_dot(cost, W_o1_ref[...]) + b_o1_ref[...])
    card = jnp.maximum(_dot(root, W_h22_ref[...]) + b_h22_ref[...], 0.0)
    card = jnp.maximum(_dot(card, W_h32_ref[...]) + b_h32_ref[...], 0.0)
    card = jax.nn.sigmoid(_dot(card, W_o2_ref[...]) + b_o2_ref[...])
    cost_ref[...] = cost
    card_ref[...] = card


@jax.jit
def kernel(op_x, feat_x, cond1_x, cond2_x, bitmap_x, has_cond,
           W_op, b_op, W_pred, b_pred, W_bm, b_bm, W_feat, b_feat,
           W_r1, b_r1, W_r2, b_r2, W_r3, b_r3,
           W_h21, b_h21, W_h31, b_h31, W_o1, b_o1,
           W_h22, b_h22, W_h32, b_h32, W_o2, b_o2):
    grid = (_B // _BT,)

    has_cond = jnp.broadcast_to(has_cond[:, :, None], (_B, _N, _HID))

    def data_spec(shape):
        blk = (_BT,) + tuple(shape[1:])
        nd = len(shape)
        return pl.BlockSpec(blk, lambda i, _nd=nd: (i,) + (0,) * (_nd - 1))

    def w_spec(shape):
        nd = len(shape)
        return pl.BlockSpec(tuple(shape), lambda i, _nd=nd: (0,) * _nd)

    b2d = lambda b: b.reshape(1, -1)
    weights = [W_op, b2d(b_op), W_pred, b2d(b_pred), W_bm, b2d(b_bm),
               W_feat, b2d(b_feat), W_r1, b2d(b_r1), W_r2, b2d(b_r2),
               W_r3, b2d(b_r3), W_h21, b2d(b_h21), W_h31, b2d(b_h31),
               W_o1, b2d(b_o1), W_h22, b2d(b_h22), W_h32, b2d(b_h32),
               W_o2, b2d(b_o2)]

    data = [op_x, feat_x, cond1_x, cond2_x, bitmap_x, has_cond]
    in_specs = [data_spec(x.shape) for x in data] + \
               [w_spec(w.shape) for w in weights]

    out_shape = (
        jax.ShapeDtypeStruct((_B, 1), jnp.float32),
        jax.ShapeDtypeStruct((_B, 1), jnp.float32),
    )
    out_specs = (
        pl.BlockSpec((_BT, 1), lambda i: (i, 0)),
        pl.BlockSpec((_BT, 1), lambda i: (i, 0)),
    )

    cost, card = pl.pallas_call(
        _tree_body,
        grid=grid,
        in_specs=in_specs,
        out_specs=out_specs,
        out_shape=out_shape,
        compiler_params=pltpu.CompilerParams(
            dimension_semantics=("arbitrary",),
        ),
    )(*data, *weights)
    return (cost, card)
